# baseline (speedup 1.0000x reference)
import functools

import jax
import jax.numpy as jnp
import numpy as np
from jax import lax
from jax.experimental import pallas as pl
from jax.experimental.pallas import tpu as pltpu
from jax.experimental.pallas import tpu_sc as plsc

ALPHA = 4
NBUF = 4


def _slow_indices(T: int) -> np.ndarray:
    n = max(1, T // ALPHA)
    return np.linspace(0.0, float(T - 1), n).astype(np.int32)


def _build_slow_gather(B, T, C, H, W, dtype, n_slow, idx):
    mesh = plsc.VectorSubcoreMesh(core_axis_name="c", subcore_axis_name="s")
    num_workers = 32
    total = B * n_slow
    per_worker = total // num_workers
    hh = H // 2
    n_chunks = per_worker * C * 2

    @functools.partial(
        pl.kernel,
        mesh=mesh,
        out_type=jax.ShapeDtypeStruct((B, n_slow, C, H, W), dtype),
        scratch_types=[
            pltpu.VMEM((NBUF, hh, W), dtype),
            pltpu.SemaphoreType.DMA,
            pltpu.SemaphoreType.DMA,
        ],
    )
    def slow_gather(in_hbm, out_hbm, buf, sem_in, sem_out):
        wid = lax.axis_index("s") * 2 + lax.axis_index("c")

        in_cp, out_cp = [], []
        for k in range(n_chunks):
            s = k // (C * 2)
            c = (k % (C * 2)) // 2
            h = k % 2
            i = wid * per_worker + s
            b = i // n_slow
            t = i % n_slow
            src_t = jnp.int32(int(idx[0]))
            for j in range(1, n_slow):
                src_t = jnp.where(t == j, jnp.int32(int(idx[j])), src_t)
            v = buf.at[k % NBUF]
            in_cp.append(pltpu.make_async_copy(
                in_hbm.at[b, src_t, c, pl.ds(h * hh, hh)], v, sem_in))
            out_cp.append(pltpu.make_async_copy(
                v, out_hbm.at[b, t, c, pl.ds(h * hh, hh)], sem_out))

        for k in range(n_chunks):
            in_cp[k].start()
            if k >= 1:
                in_cp[k - 1].wait()
                out_cp[k - 1].start()
            if k >= NBUF - 1:
                out_cp[k - (NBUF - 1)].wait()
        in_cp[n_chunks - 1].wait()
        out_cp[n_chunks - 1].start()
        for k in range(n_chunks - NBUF + 1, n_chunks):
            out_cp[k].wait()

    return slow_gather


def _build_fast_copy(shape, dtype):
    B = shape[0]

    def fast_body(in_hbm, out_hbm, sem):
        cps = [
            pltpu.make_async_copy(in_hbm.at[b], out_hbm.at[b], sem)
            for b in range(B)
        ]
        for cp in cps:
            cp.start()
        for cp in cps:
            cp.wait()

    return pl.pallas_call(
        fast_body,
        out_shape=jax.ShapeDtypeStruct(shape, dtype),
        in_specs=[pl.BlockSpec(memory_space=pl.ANY)],
        out_specs=pl.BlockSpec(memory_space=pl.ANY),
        scratch_shapes=[pltpu.SemaphoreType.DMA],
    )


def kernel(frames):
    B, T, C, H, W = frames.shape
    n_slow = max(1, T // ALPHA)
    idx = _slow_indices(T)
    slow_gather = _build_slow_gather(B, T, C, H, W, frames.dtype, n_slow, idx)
    slow_pathway = slow_gather(frames)
    fast_pathway = _build_fast_copy(frames.shape, frames.dtype)(frames)
    return (slow_pathway, fast_pathway)


# speedup vs baseline: 33.6110x; 33.6110x over previous
import functools

import jax
import jax.numpy as jnp
import numpy as np
from jax import lax
from jax.experimental import pallas as pl
from jax.experimental.pallas import tpu as pltpu
from jax.experimental.pallas import tpu_sc as plsc

ALPHA = 4
NBUF = 4


def _slow_indices(T: int) -> np.ndarray:
    n = max(1, T // ALPHA)
    return np.linspace(0.0, float(T - 1), n).astype(np.int32)


def _build_slow_gather(B, T, C, H, W, dtype, n_slow, idx):
    mesh = plsc.VectorSubcoreMesh(core_axis_name="c", subcore_axis_name="s")
    num_workers = 32
    total = B * n_slow
    per_worker = total // num_workers
    hh = H // 2
    n_chunks = per_worker * C * 2

    @functools.partial(
        pl.kernel,
        mesh=mesh,
        out_type=jax.ShapeDtypeStruct((B, n_slow, C, H, W), dtype),
        scratch_types=[
            pltpu.VMEM((NBUF, hh, W), dtype),
            pltpu.SemaphoreType.DMA,
            pltpu.SemaphoreType.DMA,
        ],
    )
    def slow_gather(in_hbm, out_hbm, buf, sem_in, sem_out):
        wid = lax.axis_index("s") * 2 + lax.axis_index("c")

        in_cp, out_cp = [], []
        for k in range(n_chunks):
            s = k // (C * 2)
            c = (k % (C * 2)) // 2
            h = k % 2
            i = wid * per_worker + s
            b = i // n_slow
            t = i % n_slow
            src_t = jnp.int32(int(idx[0]))
            for j in range(1, n_slow):
                src_t = jnp.where(t == j, jnp.int32(int(idx[j])), src_t)
            v = buf.at[k % NBUF]
            in_cp.append(pltpu.make_async_copy(
                in_hbm.at[b, src_t, c, pl.ds(h * hh, hh)], v, sem_in))
            out_cp.append(pltpu.make_async_copy(
                v, out_hbm.at[b, t, c, pl.ds(h * hh, hh)], sem_out))

        for k in range(n_chunks):
            in_cp[k].start()
            if k >= 1:
                in_cp[k - 1].wait()
                out_cp[k - 1].start()
            if k >= NBUF - 1:
                out_cp[k - (NBUF - 1)].wait()
        in_cp[n_chunks - 1].wait()
        out_cp[n_chunks - 1].start()
        for k in range(n_chunks - NBUF + 1, n_chunks):
            out_cp[k].wait()

    return slow_gather


def _build_fast_copy(shape, dtype):
    B, T, C, H, W = shape
    TB = 4

    def fast_body(in_ref, out_ref):
        out_ref[...] = in_ref[...]

    spec = pl.BlockSpec(
        (1, TB, C, H, W), lambda b, t: (b, t, 0, 0, 0))
    return pl.pallas_call(
        fast_body,
        grid=(B, T // TB),
        out_shape=jax.ShapeDtypeStruct(shape, dtype),
        in_specs=[spec],
        out_specs=spec,
    )


def kernel(frames):
    B, T, C, H, W = frames.shape
    n_slow = max(1, T // ALPHA)
    idx = _slow_indices(T)
    slow_gather = _build_slow_gather(B, T, C, H, W, frames.dtype, n_slow, idx)
    slow_pathway = slow_gather(frames)
    fast_pathway = _build_fast_copy(frames.shape, frames.dtype)(frames)
    return (slow_pathway, fast_pathway)


# speedup vs baseline: 33.8471x; 1.0070x over previous
import functools

import jax
import jax.numpy as jnp
import numpy as np
from jax import lax
from jax.experimental import pallas as pl
from jax.experimental.pallas import tpu as pltpu
from jax.experimental.pallas import tpu_sc as plsc

ALPHA = 4
NBUF = 2


def _slow_indices(T: int) -> np.ndarray:
    n = max(1, T // ALPHA)
    return np.linspace(0.0, float(T - 1), n).astype(np.int32)


def _build_pack(B, T, C, H, W, dtype, n_slow, idx):
    mesh = plsc.VectorSubcoreMesh(core_axis_name="c", subcore_axis_name="s")
    num_workers = 32
    t_per_w = (B * T) // num_workers
    w_per_b = T // t_per_w
    n_chunks = t_per_w * C

    @functools.partial(
        pl.kernel,
        mesh=mesh,
        out_type=(
            jax.ShapeDtypeStruct((B, n_slow, C, H, W), dtype),
            jax.ShapeDtypeStruct((B, T, C, H, W), dtype),
        ),
        scratch_types=[
            pltpu.VMEM((NBUF, H, W), dtype),
            pltpu.SemaphoreType.DMA,
            pltpu.SemaphoreType.DMA,
            pltpu.SemaphoreType.DMA,
        ],
    )
    def pack(in_hbm, slow_hbm, fast_hbm, buf, sem_in, sem_fast, sem_slow):
        wid = lax.axis_index("s") * 2 + lax.axis_index("c")
        b = wid // w_per_b
        tbase = (wid % w_per_b) * t_per_w

        chunks = []
        for k in range(n_chunks):
            t = tbase + (k // C)
            c = k % C
            sel = t == jnp.int32(int(idx[0]))
            j = jnp.int32(0)
            for s in idx[1:]:
                sel = jnp.logical_or(sel, t == jnp.int32(int(s)))
                j = j + jnp.where(t >= jnp.int32(int(s)), 1, 0)
            v = buf.at[k % NBUF]
            cp_in = pltpu.make_async_copy(in_hbm.at[b, t, c], v, sem_in)
            cp_fast = pltpu.make_async_copy(v, fast_hbm.at[b, t, c], sem_fast)
            cp_slow = pltpu.make_async_copy(v, slow_hbm.at[b, j, c], sem_slow)
            chunks.append((cp_in, cp_fast, cp_slow, sel))

        for k in range(n_chunks):
            if k >= NBUF:
                _, cf, cs, sel = chunks[k - NBUF]
                cf.wait()
                @pl.when(sel)
                def _(cs=cs):
                    cs.wait()
            chunks[k][0].start()
            if k >= 1:
                ci, cf, cs, sel = chunks[k - 1]
                ci.wait()
                cf.start()
                @pl.when(sel)
                def _(cs=cs):
                    cs.start()
        ci, cf, cs, sel = chunks[n_chunks - 1]
        ci.wait()
        cf.start()
        @pl.when(sel)
        def _(cs=cs):
            cs.start()
        for k in range(n_chunks - NBUF + 1, n_chunks):
            _, cf, cs, sel = chunks[k]
            cf.wait()
            @pl.when(sel)
            def _(cs=cs):
                cs.wait()

    return pack


def kernel(frames):
    B, T, C, H, W = frames.shape
    n_slow = max(1, T // ALPHA)
    idx = _slow_indices(T)
    pack = _build_pack(B, T, C, H, W, frames.dtype, n_slow, idx)
    slow_pathway, fast_pathway = pack(frames)
    return (slow_pathway, fast_pathway)


# speedup vs baseline: 34.5004x; 1.0193x over previous
import functools

import jax
import jax.numpy as jnp
import numpy as np
from jax import lax
from jax.experimental import pallas as pl
from jax.experimental.pallas import tpu as pltpu
from jax.experimental.pallas import tpu_sc as plsc

ALPHA = 4
NBUF = 4


def _slow_indices(T: int) -> np.ndarray:
    n = max(1, T // ALPHA)
    return np.linspace(0.0, float(T - 1), n).astype(np.int32)


def _build_slow_gather(B, T, C, H, W, dtype, n_slow, idx):
    mesh = plsc.VectorSubcoreMesh(core_axis_name="c", subcore_axis_name="s")
    num_workers = 32
    total = B * n_slow
    per_worker = total // num_workers
    hh = H // 2
    n_chunks = per_worker * C * 2

    @functools.partial(
        pl.kernel,
        mesh=mesh,
        out_type=jax.ShapeDtypeStruct((B, n_slow, C, H, W), dtype),
        scratch_types=[
            pltpu.VMEM((NBUF, hh, W), dtype),
            pltpu.SemaphoreType.DMA,
            pltpu.SemaphoreType.DMA,
        ],
    )
    def slow_gather(in_hbm, out_hbm, buf, sem_in, sem_out):
        wid = lax.axis_index("s") * 2 + lax.axis_index("c")

        in_cp, out_cp = [], []
        for k in range(n_chunks):
            s = k // (C * 2)
            c = (k % (C * 2)) // 2
            h = k % 2
            i = wid * per_worker + s
            b = i // n_slow
            t = i % n_slow
            src_t = jnp.int32(int(idx[0]))
            for j in range(1, n_slow):
                src_t = jnp.where(t == j, jnp.int32(int(idx[j])), src_t)
            v = buf.at[k % NBUF]
            in_cp.append(pltpu.make_async_copy(
                in_hbm.at[b, src_t, c, pl.ds(h * hh, hh)], v, sem_in))
            out_cp.append(pltpu.make_async_copy(
                v, out_hbm.at[b, t, c, pl.ds(h * hh, hh)], sem_out))

        for k in range(n_chunks):
            in_cp[k].start()
            if k >= 1:
                in_cp[k - 1].wait()
                out_cp[k - 1].start()
            if k >= NBUF - 1:
                out_cp[k - (NBUF - 1)].wait()
        in_cp[n_chunks - 1].wait()
        out_cp[n_chunks - 1].start()
        for k in range(n_chunks - NBUF + 1, n_chunks):
            out_cp[k].wait()

    return slow_gather


def _build_fast_copy(shape, dtype):
    B, T, C, H, W = shape
    TB = 8

    def fast_body(in_ref, out_ref):
        out_ref[...] = in_ref[...]

    spec = pl.BlockSpec(
        (1, TB, C, H, W), lambda b, t: (b, t, 0, 0, 0))
    return pl.pallas_call(
        fast_body,
        grid=(B, T // TB),
        out_shape=jax.ShapeDtypeStruct(shape, dtype),
        in_specs=[spec],
        out_specs=spec,
    )


def kernel(frames):
    B, T, C, H, W = frames.shape
    n_slow = max(1, T // ALPHA)
    idx = _slow_indices(T)
    slow_gather = _build_slow_gather(B, T, C, H, W, frames.dtype, n_slow, idx)
    slow_pathway = slow_gather(frames)
    fast_pathway = _build_fast_copy(frames.shape, frames.dtype)(frames)
    return (slow_pathway, fast_pathway)


# speedup vs baseline: 34.5900x; 1.0026x over previous
import functools

import jax
import jax.numpy as jnp
import numpy as np
from jax import lax
from jax.experimental import pallas as pl
from jax.experimental.pallas import tpu as pltpu
from jax.experimental.pallas import tpu_sc as plsc

ALPHA = 4
NBUF = 4


def _slow_indices(T: int) -> np.ndarray:
    n = max(1, T // ALPHA)
    return np.linspace(0.0, float(T - 1), n).astype(np.int32)


def _build_slow_gather(B, T, C, H, W, dtype, n_slow, idx):
    mesh = plsc.VectorSubcoreMesh(core_axis_name="c", subcore_axis_name="s")
    num_workers = 32
    total = B * n_slow
    per_worker = total // num_workers
    hh = H // 2
    n_chunks = per_worker * C * 2

    @functools.partial(
        pl.kernel,
        mesh=mesh,
        out_type=jax.ShapeDtypeStruct((B, n_slow, C, H, W), dtype),
        scratch_types=[
            pltpu.VMEM((NBUF, hh, W), dtype),
            pltpu.SemaphoreType.DMA,
            pltpu.SemaphoreType.DMA,
        ],
    )
    def slow_gather(in_hbm, out_hbm, buf, sem_in, sem_out):
        wid = lax.axis_index("s") * 2 + lax.axis_index("c")

        in_cp, out_cp = [], []
        for k in range(n_chunks):
            s = k // (C * 2)
            c = (k % (C * 2)) // 2
            h = k % 2
            i = wid * per_worker + s
            b = i // n_slow
            t = i % n_slow
            src_t = jnp.int32(int(idx[0]))
            for j in range(1, n_slow):
                src_t = jnp.where(t == j, jnp.int32(int(idx[j])), src_t)
            v = buf.at[k % NBUF]
            in_cp.append(pltpu.make_async_copy(
                in_hbm.at[b, src_t, c, pl.ds(h * hh, hh)], v, sem_in))
            out_cp.append(pltpu.make_async_copy(
                v, out_hbm.at[b, t, c, pl.ds(h * hh, hh)], sem_out))

        for k in range(n_chunks):
            in_cp[k].start()
            if k >= 1:
                in_cp[k - 1].wait()
                out_cp[k - 1].start()
            if k >= NBUF - 1:
                out_cp[k - (NBUF - 1)].wait()
        in_cp[n_chunks - 1].wait()
        out_cp[n_chunks - 1].start()
        for k in range(n_chunks - NBUF + 1, n_chunks):
            out_cp[k].wait()

    return slow_gather


def _build_fast_copy(shape, dtype):
    B, T, C, H, W = shape
    TB = 8
    nt = T // TB
    n = B * nt

    FBUF = 4
    chunks = [(b, t) for b in range(B) for t in range(0, T, TB)]

    def fast_body(in_hbm, out_hbm, buf, sem_in, sem_out):
        cps = []
        for k, (b, t) in enumerate(chunks):
            v = buf.at[k % FBUF]
            cps.append((
                pltpu.make_async_copy(
                    in_hbm.at[b, pl.ds(t, TB)], v, sem_in.at[k % FBUF]),
                pltpu.make_async_copy(
                    v, out_hbm.at[b, pl.ds(t, TB)], sem_out.at[k % FBUF]),
            ))
        for k in range(n):
            if k >= FBUF:
                cps[k - FBUF][1].wait()
            cps[k][0].start()
            if k >= 1:
                cps[k - 1][0].wait()
                cps[k - 1][1].start()
        cps[n - 1][0].wait()
        cps[n - 1][1].start()
        for k in range(n - FBUF, n):
            cps[k][1].wait()

    return pl.pallas_call(
        fast_body,
        out_shape=jax.ShapeDtypeStruct(shape, dtype),
        in_specs=[pl.BlockSpec(memory_space=pl.ANY)],
        out_specs=pl.BlockSpec(memory_space=pl.ANY),
        scratch_shapes=[
            pltpu.VMEM((FBUF, TB, C, H, W), dtype),
            pltpu.SemaphoreType.DMA((FBUF,)),
            pltpu.SemaphoreType.DMA((FBUF,)),
        ],
    )


def kernel(frames):
    B, T, C, H, W = frames.shape
    n_slow = max(1, T // ALPHA)
    idx = _slow_indices(T)
    slow_gather = _build_slow_gather(B, T, C, H, W, frames.dtype, n_slow, idx)
    slow_pathway = slow_gather(frames)
    fast_pathway = _build_fast_copy(frames.shape, frames.dtype)(frames)
    return (slow_pathway, fast_pathway)


# speedup vs baseline: 34.6901x; 1.0029x over previous
import functools

import jax
import jax.numpy as jnp
import numpy as np
from jax import lax
from jax.experimental import pallas as pl
from jax.experimental.pallas import tpu as pltpu
from jax.experimental.pallas import tpu_sc as plsc

ALPHA = 4
NBUF = 4


def _slow_indices(T: int) -> np.ndarray:
    n = max(1, T // ALPHA)
    return np.linspace(0.0, float(T - 1), n).astype(np.int32)


def _build_slow_gather(B, T, C, H, W, dtype, n_slow, idx):
    mesh = plsc.VectorSubcoreMesh(core_axis_name="c", subcore_axis_name="s")
    num_workers = 32
    total = B * n_slow
    per_worker = total // num_workers
    hh = H // 2
    n_chunks = per_worker * C * 2

    @functools.partial(
        pl.kernel,
        mesh=mesh,
        out_type=jax.ShapeDtypeStruct((B, n_slow, C, H, W), dtype),
        scratch_types=[
            pltpu.VMEM((NBUF, hh, W), dtype),
            pltpu.SemaphoreType.DMA,
            pltpu.SemaphoreType.DMA,
        ],
    )
    def slow_gather(in_hbm, out_hbm, buf, sem_in, sem_out):
        wid = lax.axis_index("s") * 2 + lax.axis_index("c")

        in_cp, out_cp = [], []
        for k in range(n_chunks):
            s = k // (C * 2)
            c = (k % (C * 2)) // 2
            h = k % 2
            i = wid * per_worker + s
            b = i // n_slow
            t = i % n_slow
            src_t = jnp.int32(int(idx[0]))
            for j in range(1, n_slow):
                src_t = jnp.where(t == j, jnp.int32(int(idx[j])), src_t)
            v = buf.at[k % NBUF]
            in_cp.append(pltpu.make_async_copy(
                in_hbm.at[b, src_t, c, pl.ds(h * hh, hh)], v, sem_in))
            out_cp.append(pltpu.make_async_copy(
                v, out_hbm.at[b, t, c, pl.ds(h * hh, hh)], sem_out))

        for k in range(n_chunks):
            in_cp[k].start()
            if k >= 1:
                in_cp[k - 1].wait()
                out_cp[k - 1].start()
            if k >= NBUF - 1:
                out_cp[k - (NBUF - 1)].wait()
        in_cp[n_chunks - 1].wait()
        out_cp[n_chunks - 1].start()
        for k in range(n_chunks - NBUF + 1, n_chunks):
            out_cp[k].wait()

    return slow_gather


def _build_fast_copy(shape, dtype):
    B, T, C, H, W = shape
    TB = 16
    nt = T // TB
    n = B * nt

    FBUF = 4
    chunks = [(b, t) for b in range(B) for t in range(0, T, TB)]

    def fast_body(in_hbm, out_hbm, buf, sem_in, sem_out):
        cps = []
        for k, (b, t) in enumerate(chunks):
            v = buf.at[k % FBUF]
            cps.append((
                pltpu.make_async_copy(
                    in_hbm.at[b, pl.ds(t, TB)], v, sem_in.at[k % FBUF]),
                pltpu.make_async_copy(
                    v, out_hbm.at[b, pl.ds(t, TB)], sem_out.at[k % FBUF]),
            ))
        for k in range(n):
            if k >= FBUF:
                cps[k - FBUF][1].wait()
            cps[k][0].start()
            if k >= 1:
                cps[k - 1][0].wait()
                cps[k - 1][1].start()
        cps[n - 1][0].wait()
        cps[n - 1][1].start()
        for k in range(n - FBUF, n):
            cps[k][1].wait()

    return pl.pallas_call(
        fast_body,
        out_shape=jax.ShapeDtypeStruct(shape, dtype),
        in_specs=[pl.BlockSpec(memory_space=pl.ANY)],
        out_specs=pl.BlockSpec(memory_space=pl.ANY),
        scratch_shapes=[
            pltpu.VMEM((FBUF, TB, C, H, W), dtype),
            pltpu.SemaphoreType.DMA((FBUF,)),
            pltpu.SemaphoreType.DMA((FBUF,)),
        ],
    )


def kernel(frames):
    B, T, C, H, W = frames.shape
    n_slow = max(1, T // ALPHA)
    idx = _slow_indices(T)
    slow_gather = _build_slow_gather(B, T, C, H, W, frames.dtype, n_slow, idx)
    slow_pathway = slow_gather(frames)
    fast_pathway = _build_fast_copy(frames.shape, frames.dtype)(frames)
    return (slow_pathway, fast_pathway)
